# Initial kernel scaffold; baseline (speedup 1.0000x reference)
import jax
import jax.numpy as jnp
from jax.experimental import pallas as pl


def kernel(x, skip, wt, bt, w1, g1, b1, w2, g2, b2):
    raise NotImplementedError("write your pallas kernel here")


# speedup vs baseline: 4.7616x; 4.7616x over previous
import functools

import jax
import jax.numpy as jnp
from jax.experimental import pallas as pl
from jax.experimental.pallas import tpu as pltpu

_EPS = 1e-5
_BF = jnp.bfloat16

_D = 32
_PW = _D + 2
_PLANE = _PW * _PW
_ROWS = _D * _PLANE
_CHUNK = 4 * _PLANE
_NCHUNK = _ROWS // _CHUNK
_XROWS = 9 * _CHUNK
_N = 4
_M_TOTAL = _N * _D * _D * _D
_OFFS = tuple(kd * _PLANE + kh * _PW + kw
              for kd in range(3) for kh in range(3) for kw in range(3))

_VMEM = 56 * 1024 * 1024


def _convt_kernel(x_ref, w_ref, b_ref, o_ref):
    acc = jnp.dot(x_ref[...].astype(_BF), w_ref[...],
                  preferred_element_type=jnp.float32)
    o_ref[...] = acc + b_ref[...]


def _conv_stats_kernel(xlo_ref, xhi_ref, w_ref, m_ref, y_ref, st_ref,
                       cat_ref, cols_ref, *, cin):
    cat_ref[0:_CHUNK, :] = xlo_ref[0]
    cat_ref[_CHUNK:, :] = xhi_ref[0]
    for t, off in enumerate(_OFFS):
        cols_ref[:, cin * t:cin * (t + 1)] = (
            cat_ref[off:off + _CHUNK, :].astype(_BF))
    y = jnp.dot(cols_ref[...], w_ref[...], preferred_element_type=jnp.float32)
    y_ref[0] = y
    ym = y * m_ref[...]
    s = jnp.sum(ym, axis=0, keepdims=True)
    q = jnp.sum(ym * ym, axis=0, keepdims=True)
    st_ref[...] = jnp.concatenate([s, q], axis=0).reshape(st_ref.shape)


def _affine_kernel(y_ref, sc_ref, sh_ref, m_ref, o_ref, *, masked):
    z = jnp.maximum(y_ref[...] * sc_ref[...] + sh_ref[...], 0.0)
    if masked:
        z = z * m_ref[...]
    o_ref[...] = z


def _convt(x, wt, bt):
    n, cin, d, h, w = x.shape
    cout = wt.shape[1]
    m = n * d * h * w
    xt = jnp.transpose(x, (0, 2, 3, 4, 1)).reshape(m, cin)
    wmat = jnp.transpose(wt, (0, 2, 3, 4, 1)).reshape(cin, 8 * cout).astype(_BF)
    bmat = jnp.tile(bt, 8).reshape(1, 8 * cout)
    tm = m // 4
    y = pl.pallas_call(
        _convt_kernel,
        out_shape=jax.ShapeDtypeStruct((m, 8 * cout), jnp.float32),
        grid=(4,),
        in_specs=[pl.BlockSpec((tm, cin), lambda i: (i, 0)),
                  pl.BlockSpec((cin, 8 * cout), lambda i: (0, 0)),
                  pl.BlockSpec((1, 8 * cout), lambda i: (0, 0))],
        out_specs=pl.BlockSpec((tm, 8 * cout), lambda i: (i, 0)),
        compiler_params=pltpu.CompilerParams(
            dimension_semantics=("parallel",), vmem_limit_bytes=_VMEM),
    )(xt, wmat, bmat)
    y = y.reshape(n, d, h, w, 2, 2, 2, cout)
    y = jnp.transpose(y, (0, 1, 4, 2, 5, 3, 6, 7))
    return y.reshape(n, 2 * d, 2 * h, 2 * w, cout)


def _conv_block(xp, wmat, mask_rows, cin):
    k = 27 * cin
    y, st = pl.pallas_call(
        functools.partial(_conv_stats_kernel, cin=cin),
        out_shape=(jax.ShapeDtypeStruct((_N, _ROWS, 32), jnp.float32),
                   jax.ShapeDtypeStruct((_N, _NCHUNK, 2, 32), jnp.float32)),
        grid=(_N, _NCHUNK),
        in_specs=[pl.BlockSpec((1, _CHUNK, cin), lambda n, d: (n, d, 0)),
                  pl.BlockSpec((1, _CHUNK, cin), lambda n, d: (n, d + 1, 0)),
                  pl.BlockSpec((k, 32), lambda n, d: (0, 0)),
                  pl.BlockSpec((_CHUNK, 32), lambda n, d: (0, 0))],
        out_specs=[pl.BlockSpec((1, _CHUNK, 32), lambda n, d: (n, d, 0)),
                   pl.BlockSpec((1, 1, 2, 32), lambda n, d: (n, d, 0, 0))],
        scratch_shapes=[pltpu.VMEM((2 * _CHUNK, cin), jnp.float32),
                        pltpu.VMEM((_CHUNK, k), _BF)],
        compiler_params=pltpu.CompilerParams(
            dimension_semantics=("parallel", "arbitrary"),
            vmem_limit_bytes=_VMEM),
    )(xp, xp, wmat, mask_rows)
    return y, st


def _bn_affine(y_raw, st, gamma, beta, mask_lanes, masked):
    ssum = st.sum(axis=(0, 1))
    mean = ssum[0] / _M_TOTAL
    var = ssum[1] / _M_TOTAL - mean * mean
    scale = gamma * jax.lax.rsqrt(var + _EPS)
    shift = beta - mean * scale
    sc = jnp.tile(scale, 4).reshape(1, 128)
    sh = jnp.tile(shift, 4).reshape(1, 128)
    yl = y_raw.reshape(_N * _ROWS // 4, 128)
    rl = yl.shape[0]
    tr = rl // 8
    out = pl.pallas_call(
        functools.partial(_affine_kernel, masked=masked),
        out_shape=jax.ShapeDtypeStruct((rl, 128), jnp.float32),
        grid=(8,),
        in_specs=[pl.BlockSpec((tr, 128), lambda i: (i, 0)),
                  pl.BlockSpec((1, 128), lambda i: (0, 0)),
                  pl.BlockSpec((1, 128), lambda i: (0, 0)),
                  pl.BlockSpec((tr, 128), lambda i: (i % 2, 0))],
        out_specs=pl.BlockSpec((tr, 128), lambda i: (i, 0)),
        compiler_params=pltpu.CompilerParams(
            dimension_semantics=("parallel",), vmem_limit_bytes=_VMEM),
    )(yl, sc, sh, mask_lanes)
    return out


def _build_masks():
    p = jnp.arange(_CHUNK, dtype=jnp.int32) % _PLANE
    valid = ((p // _PW) < _D) & ((p % _PW) < _D)
    mask_rows = jnp.broadcast_to(
        valid[:, None].astype(jnp.float32), (_CHUNK, 32))
    rr = jnp.arange(_ROWS * 32 // 128, dtype=jnp.int32)
    gr = rr[:, None] * 4 + jnp.arange(128, dtype=jnp.int32)[None, :] // 32
    pp = gr % _PLANE
    vl = ((pp // _PW) < _D) & ((pp % _PW) < _D)
    return mask_rows, vl.astype(jnp.float32)


def _pad_cat(v):
    vp = jnp.pad(v, ((0, 0), (1, 1), (1, 1), (1, 1), (0, 0)))
    c = v.shape[-1]
    vp = vp.reshape(_N, _PW * _PLANE, c)
    return jnp.pad(vp, ((0, 0), (0, _XROWS - _PW * _PLANE), (0, 0)))


def _pad_valid(ya):
    v = ya.reshape(_N, _ROWS, 32)
    lead = _PLANE + _PW + 1
    return jnp.pad(v, ((0, 0), (lead, _XROWS - _ROWS - lead), (0, 0)))


def kernel(x, skip, wt, bt, w1, g1, b1, w2, g2, b2):
    mask_rows, mask_lanes = _build_masks()
    xu = _convt(x, wt, bt)
    sk = jnp.transpose(skip[:, :, 2:34, 2:34, 2:34], (0, 2, 3, 4, 1))
    cat = jnp.concatenate([xu, sk], axis=-1)
    x1 = _pad_cat(cat)
    w1m = w1.reshape(27 * 64, 32).astype(_BF)
    y1, st1 = _conv_block(x1, w1m, mask_rows, 64)
    y1a = _bn_affine(y1, st1, g1, b1, mask_lanes, masked=True)
    x2 = _pad_valid(y1a)
    w2m = w2.reshape(27 * 32, 32).astype(_BF)
    y2, st2 = _conv_block(x2, w2m, mask_rows, 32)
    y2a = _bn_affine(y2, st2, g2, b2, mask_lanes, masked=False)
    out = y2a.reshape(_N, _D, _PW, _PW, 32)[:, :, :_D, :_D, :]
    return jnp.transpose(out, (0, 4, 1, 2, 3))
